# Initial kernel scaffold; baseline (speedup 1.0000x reference)
import jax
import jax.numpy as jnp
from jax.experimental import pallas as pl


def kernel(q, k, v):
    raise NotImplementedError("write your pallas kernel here")


# speedup vs baseline: 1.4742x; 1.4742x over previous
import math

import jax
import jax.numpy as jnp
from jax.experimental import pallas as pl

_B, _H, _S, _D = 1, 16, 2048, 128
_BM = 192
_TOPK = 512
_RAND_P = 0.01
_LOCAL_W = 128
_G = -(-_S // _BM)
_SP = _G * _BM
_CH = 256


def _fused_kernel(q_ref, k_ref, v_ref, rm_ref, o_ref, oc_ref):
    g = pl.program_id(1)
    q = q_ref[0, 0]
    k = k_ref[0, 0]
    v = v_ref[0, 0]
    scale = 1.0 / math.sqrt(_D)

    logits = jax.lax.dot_general(
        q.astype(jnp.bfloat16), k.astype(jnp.bfloat16),
        (((1,), (1,)), ((), ())),
        preferred_element_type=jnp.float32) * scale
    m = jnp.max(logits, axis=-1, keepdims=True)
    p = jnp.exp(logits - m)
    l = jnp.sum(p, axis=-1, keepdims=True)
    vb = v.astype(jnp.bfloat16)
    o = jax.lax.dot_general(
        p.astype(jnp.bfloat16), vb, (((1,), (0,)), ((), ())),
        preferred_element_type=jnp.float32) / l
    o_ref[0, 0] = o

    row = jax.lax.broadcasted_iota(jnp.int32, (_BM, 1), 0)
    valid = (g * _BM + row) < _S
    probs = (p / l).astype(jnp.bfloat16).astype(jnp.float32)
    probs = jnp.where(valid, probs, 0.0)
    bs = jnp.sum(probs, axis=0, keepdims=True)

    kidx = jax.lax.broadcasted_iota(jnp.int32, (1, _S), 1)
    rank = jnp.zeros((1, _S), dtype=jnp.float32)
    for c in range(_S // _CH):
        rsel = jax.lax.broadcasted_iota(jnp.int32, (_CH, _S), 0)
        csel = jax.lax.broadcasted_iota(jnp.int32, (_CH, _S), 1)
        sel = (csel == rsel + c * _CH).astype(jnp.float32)
        bsi = jax.lax.dot_general(
            sel, bs, (((1,), (1,)), ((), ())),
            precision=jax.lax.Precision.HIGHEST,
            preferred_element_type=jnp.float32)
        ii = c * _CH + jax.lax.broadcasted_iota(jnp.int32, (_CH, 1), 0)
        beat = (bsi > bs) | ((bsi == bs) & (ii < kidx))
        rank = rank + jnp.sum(beat.astype(jnp.float32), axis=0, keepdims=True)
    topk_mask = rank < float(_TOPK)

    gstart = g * _BM
    static = (kidx >= gstart - _LOCAL_W) & (kidx < gstart + _BM + _LOCAL_W)
    rmask = rm_ref[0, 0] != 0
    mask = topk_mask | rmask | static

    masked = jnp.where(mask, logits, -1e30)
    m2 = jnp.max(masked, axis=-1, keepdims=True)
    p2 = jnp.exp(masked - m2)
    l2 = jnp.sum(p2, axis=-1, keepdims=True)
    o_sparse = jax.lax.dot_general(
        p2.astype(jnp.bfloat16), vb, (((1,), (0,)), ((), ())),
        preferred_element_type=jnp.float32) / l2
    oc_ref[0, 0] = o - o_sparse


def kernel(q, k, v):
    b, h, s, d = q.shape
    rkey = jax.random.fold_in(jax.random.key(1), 7)
    rmask = (jax.random.uniform(rkey, (b, h, _G, s)) < _RAND_P)
    rmask = rmask[0].reshape(h, _G, 1, s).astype(jnp.int32)

    qp = jnp.pad(q, ((0, 0), (0, 0), (0, _SP - s), (0, 0)))

    o, oc = pl.pallas_call(
        _fused_kernel,
        grid=(h, _G),
        in_specs=[
            pl.BlockSpec((1, 1, _BM, d), lambda hh, gg: (0, hh, gg, 0)),
            pl.BlockSpec((1, 1, s, d), lambda hh, gg: (0, hh, 0, 0)),
            pl.BlockSpec((1, 1, s, d), lambda hh, gg: (0, hh, 0, 0)),
            pl.BlockSpec((1, 1, 1, s), lambda hh, gg: (hh, gg, 0, 0)),
        ],
        out_specs=[
            pl.BlockSpec((1, 1, _BM, d), lambda hh, gg: (0, hh, gg, 0)),
            pl.BlockSpec((1, 1, _BM, d), lambda hh, gg: (0, hh, gg, 0)),
        ],
        out_shape=[
            jax.ShapeDtypeStruct((b, h, _SP, d), jnp.float32),
            jax.ShapeDtypeStruct((b, h, _SP, d), jnp.float32),
        ],
    )(qp, k, v, rmask)

    return jnp.stack([o[:, :, :s], oc[:, :, :s]], axis=0)
